# Initial kernel scaffold; baseline (speedup 1.0000x reference)
import jax
import jax.numpy as jnp
from jax.experimental import pallas as pl


def kernel(z, pos, edge_index, batch, params):
    raise NotImplementedError("write your pallas kernel here")


# speedup vs baseline: 10.3578x; 10.3578x over previous
import functools

import jax
import jax.numpy as jnp
from jax.experimental import pallas as pl
from jax.experimental.pallas import tpu as pltpu

H = 64
VD = 32
NG = 64
CUTOFF = 5.0

BE = 4000
BN = 1024


def _silu(x):
    return x * jax.nn.sigmoid(x)


def _edge_body(hj, hi, W1a, W1b, w1c, b1, W2, b2, out):
    hjh = hj[:, :H]
    hih = hi[:, :H]
    rx = hj[:, H:H + 1] - hi[:, H:H + 1]
    ry = hj[:, H + 1:H + 2] - hi[:, H + 1:H + 2]
    rz = hj[:, H + 2:H + 3] - hi[:, H + 2:H + 3]
    d2 = rx * rx + ry * ry + rz * rz + 1e-12
    dist = jnp.sqrt(d2)
    t = hih @ W1a[...] + hjh @ W1b[...] + dist * w1c[...] + b1[...]
    t = _silu(t)
    m = t @ W2[...] + b2[...]
    x = jnp.clip(dist / CUTOFF, 0.0, 1.0)
    w = 1.0 - 6.0 * x ** 5 + 15.0 * x ** 4 - 10.0 * x ** 3
    m = m * w
    inv = 1.0 / (dist + 1e-8)
    m32 = m[:, :VD]
    out[...] = jnp.concatenate(
        [m, (rx * inv) * m32, (ry * inv) * m32, (rz * inv) * m32], axis=1)


def _node_body(h, V, hm, vm,
               nW1, nb1, nW2, nb2,
               gW1, gb1, gW2, gb2,
               mW1a, mW1b, mb1, mW2, mb2,
               qW1, qb1, qW2, qb2,
               h_out, V_out):
    hmv = hm[...]
    t = _silu(hmv @ nW1[...] + nb1[...])
    hu = t @ nW2[...] + nb2[...]
    t = _silu(hmv @ gW1[...] + gb1[...])
    gate = t @ gW2[...] + gb2[...]
    sg = jax.nn.sigmoid(gate)
    sg3 = jnp.concatenate([sg, sg, sg], axis=1)
    Vn = V[...] + vm[...] * sg3
    hn = h[...] + hu
    Vx = Vn[:, :VD]
    Vy = Vn[:, VD:2 * VD]
    Vz = Vn[:, 2 * VD:]
    vnorm = jnp.sqrt(Vx * Vx + Vy * Vy + Vz * Vz + 1e-12)
    t = _silu(hn @ mW1a[...] + vnorm @ mW1b[...] + mb1[...])
    hn = hn + (t @ mW2[...] + mb2[...])
    t = _silu(hn @ qW1[...] + qb1[...])
    g2 = t @ qW2[...] + qb2[...]
    sq = jax.nn.sigmoid(g2)
    V_out[...] = Vn * jnp.concatenate([sq, sq, sq], axis=1)
    h_out[...] = hn


def _readout_body(h, bb, oW1, ob1, oW2, ob2, atom_out, mol_out, acc):
    i = pl.program_id(0)
    t = _silu(h[...] @ oW1[...] + ob1[...])
    a8 = t @ oW2[...] + ob2[...]
    atom_out[...] = a8[:, 0:1]
    lanes = jax.lax.broadcasted_iota(jnp.int32, (BN, NG), 1)
    oh = (bb[...] == lanes).astype(jnp.float32)
    stat = jnp.concatenate(
        [a8[:, 0:1], jnp.ones((BN, 1), jnp.float32),
         jnp.zeros((BN, 6), jnp.float32)], axis=1)
    part = jax.lax.dot_general(oh, stat, (((0,), (0,)), ((), ())))

    @pl.when(i == 0)
    def _():
        acc[...] = part

    @pl.when(i != 0)
    def _():
        acc[...] = acc[...] + part

    @pl.when(i == pl.num_programs(0) - 1)
    def _():
        a = acc[...]
        mol_out[...] = a[:, 0:1] / jnp.clip(a[:, 1:2], 1.0, None)


def _const_spec(shape):
    return pl.BlockSpec(shape, lambda i: (0,) * len(shape))


def _edge_call(hj80, hi80, ew):
    EP = hj80.shape[0]
    W1, b1 = ew[0]
    W2, b2 = ew[1]
    specs = [
        pl.BlockSpec((BE, 80), lambda i: (i, 0)),
        pl.BlockSpec((BE, 80), lambda i: (i, 0)),
        _const_spec((H, H)), _const_spec((H, H)), _const_spec((1, H)),
        _const_spec((1, H)), _const_spec((H, H)), _const_spec((1, H)),
    ]
    return pl.pallas_call(
        _edge_body,
        grid=(EP // BE,),
        in_specs=specs,
        out_specs=pl.BlockSpec((BE, 160), lambda i: (i, 0)),
        out_shape=jax.ShapeDtypeStruct((EP, 160), jnp.float32),
    )(hj80, hi80, W1[:H], W1[H:2 * H], W1[2 * H:2 * H + 1], b1[None, :],
      W2, b2[None, :])


def _node_call(hp, Vp, hmp, vmp, mpp, mixp):
    NP = hp.shape[0]
    nW1, nb1 = mpp['node'][0]
    nW2, nb2 = mpp['node'][1]
    gW1, gb1 = mpp['vgate'][0]
    gW2, gb2 = mpp['vgate'][1]
    mW1, mb1 = mixp['norm'][0]
    mW2, mb2 = mixp['norm'][1]
    qW1, qb1 = mixp['gate'][0]
    qW2, qb2 = mixp['gate'][1]
    dspec = lambda w: pl.BlockSpec((BN, w), lambda i: (i, 0))
    specs = [dspec(H), dspec(96), dspec(H), dspec(96),
             _const_spec((H, H)), _const_spec((1, H)), _const_spec((H, H)), _const_spec((1, H)),
             _const_spec((H, H)), _const_spec((1, H)), _const_spec((H, VD)), _const_spec((1, VD)),
             _const_spec((H, H)), _const_spec((VD, H)), _const_spec((1, H)),
             _const_spec((H, H)), _const_spec((1, H)),
             _const_spec((H, H)), _const_spec((1, H)), _const_spec((H, VD)), _const_spec((1, VD))]
    return pl.pallas_call(
        _node_body,
        grid=(NP // BN,),
        in_specs=specs,
        out_specs=[dspec(H), dspec(96)],
        out_shape=[jax.ShapeDtypeStruct((NP, H), jnp.float32),
                   jax.ShapeDtypeStruct((NP, 96), jnp.float32)],
    )(hp, Vp, hmp, vmp,
      nW1, nb1[None, :], nW2, nb2[None, :],
      gW1, gb1[None, :], gW2, gb2[None, :],
      mW1[:H], mW1[H:], mb1[None, :], mW2, mb2[None, :],
      qW1, qb1[None, :], qW2, qb2[None, :])


def _readout_call(hp, bbp, ow):
    NP = hp.shape[0]
    oW1, ob1 = ow[0]
    oW2, ob2 = ow[1]
    oW2p = jnp.pad(oW2, ((0, 0), (0, 7)))
    ob2p = jnp.pad(ob2, (0, 7))[None, :]
    return pl.pallas_call(
        _readout_body,
        grid=(NP // BN,),
        in_specs=[pl.BlockSpec((BN, H), lambda i: (i, 0)),
                  pl.BlockSpec((BN, 1), lambda i: (i, 0)),
                  _const_spec((H, H)), _const_spec((1, H)),
                  _const_spec((H, 8)), _const_spec((1, 8))],
        out_specs=[pl.BlockSpec((BN, 1), lambda i: (i, 0)),
                   pl.BlockSpec((NG, 1), lambda i: (0, 0))],
        out_shape=[jax.ShapeDtypeStruct((NP, 1), jnp.float32),
                   jax.ShapeDtypeStruct((NG, 1), jnp.float32)],
        scratch_shapes=[pltpu.VMEM((NG, 8), jnp.float32)],
    )(hp, bbp, oW1, ob1[None, :], oW2p, ob2p)


def kernel(z, pos, edge_index, batch, params):
    N = z.shape[0]
    E = edge_index.shape[1]
    NP = -(-N // BN) * BN
    EP = -(-E // BE) * BE
    src = edge_index[0]
    dst = edge_index[1]

    h = params['emb'][z]
    V = jnp.zeros((N, 96), jnp.float32)
    posp = jnp.pad(pos, ((0, 0), (0, 13)))

    for mpp, mixp in zip(params['mp'], params['mix']):
        T = jnp.concatenate([h, posp], axis=1)
        hj80 = jnp.pad(T[src], ((0, EP - E), (0, 0)))
        hi80 = jnp.pad(T[dst], ((0, EP - E), (0, 0)))
        eout = _edge_call(hj80, hi80, mpp['edge'])[:E]
        hm = jax.ops.segment_sum(eout[:, :H], dst, num_segments=N)
        vm = jax.ops.segment_sum(eout[:, H:], dst, num_segments=N)
        hp, Vp = _node_call(
            jnp.pad(h, ((0, NP - N), (0, 0))),
            jnp.pad(V, ((0, NP - N), (0, 0))),
            jnp.pad(hm, ((0, NP - N), (0, 0))),
            jnp.pad(vm, ((0, NP - N), (0, 0))),
            mpp, mixp)
        h, V = hp[:N], Vp[:N]

    bbp = jnp.pad(batch.astype(jnp.int32), (0, NP - N),
                  constant_values=NG)[:, None]
    atomp, mol = _readout_call(jnp.pad(h, ((0, NP - N), (0, 0))), bbp,
                               params['out'])
    return (mol, atomp[:N])


# speedup vs baseline: 15.8329x; 1.5286x over previous
import functools

import jax
import jax.numpy as jnp
from jax import lax
from jax.experimental import pallas as pl
from jax.experimental.pallas import tpu as pltpu
from jax.experimental.pallas import tpu_sc as plsc

H = 64
VD = 32
NG = 64
CUTOFF = 5.0

BE = 4000
BN = 1024

NC = 2
NS = 16
NW = NC * NS
CH = 128


def _silu(x):
    return x * jax.nn.sigmoid(x)


def _mesh():
    return plsc.VectorSubcoreMesh(core_axis_name="c", subcore_axis_name="s",
                                  num_cores=NC, num_subcores=NS)


@functools.lru_cache(maxsize=None)
def _make_gather(TR, NI, D):
    W = NI // NW
    full, tail = divmod(W, CH)

    @functools.partial(
        pl.kernel,
        out_type=jax.ShapeDtypeStruct((NI, D), jnp.float32),
        mesh=_mesh(),
        compiler_params=pltpu.CompilerParams(use_tc_tiling_on_sc=False),
        scratch_types=[
            pltpu.VMEM((CH,), jnp.int32),
            pltpu.VMEM((CH,), jnp.int32),
            pltpu.VMEM((CH, D), jnp.float32),
            pltpu.SemaphoreType.DMA,
            pltpu.SemaphoreType.DMA,
            pltpu.SemaphoreType.DMA,
        ],
    )
    def gath(tab, eidx, out, idx0, idx1, rows, semi0, semi1, semg):
        c = lax.axis_index("c")
        s = lax.axis_index("s")
        base = (s * NC + c) * W
        ibufs = (idx0, idx1)
        isems = (semi0, semi1)

        def istart(g, k, n):
            pltpu.async_copy(eidx.at[pl.ds(base + g * CH, n)],
                             ibufs[k].at[pl.ds(0, n)], isems[k])

        def iwait(g, k, n):
            pltpu.make_async_copy(eidx.at[pl.ds(base + g * CH, n)],
                                  ibufs[k].at[pl.ds(0, n)], isems[k]).wait()

        def do_chunk(g, k, n):
            iwait(g, k, n)
            pltpu.async_copy(tab.at[ibufs[k].at[pl.ds(0, n)]],
                             rows.at[pl.ds(0, n), :], semg).wait()
            pltpu.sync_copy(rows.at[pl.ds(0, n), :],
                            out.at[pl.ds(base + g * CH, n), :])

        if full > 0:
            istart(0, 0, CH)

            def outer(g0, _):
                for b in range(2):
                    g = g0 + b
                    @pl.when(g < full)
                    def _():
                        nxt = g + 1
                        @pl.when(nxt < full)
                        def _():
                            istart(nxt, (b + 1) % 2, CH)
                        @pl.when(jnp.logical_and(nxt == full, tail > 0))
                        def _():
                            istart(nxt, (b + 1) % 2, max(tail, 1))
                        do_chunk(g, b, CH)
                return _

            lax.fori_loop(0, (full + 1) // 2, lambda i, u: outer(2 * i, u),
                          None)
            if tail > 0:
                do_chunk(full, full % 2, tail)
        else:
            if tail > 0:
                istart(0, 0, tail)
                do_chunk(0, 0, tail)

    return gath


@functools.lru_cache(maxsize=None)
def _make_scatter(EP, NPAD):
    W = EP // NW
    full, tail = divmod(W, CH)
    RPT = NPAD // NS
    EPC = EP // NC

    odt = [jax.ShapeDtypeStruct((NC, NPAD, VD), jnp.float32)
           for _ in range(5)]

    @functools.partial(
        pl.kernel,
        out_type=odt,
        mesh=_mesh(),
        compiler_params=pltpu.CompilerParams(use_tc_tiling_on_sc=False),
        scratch_types=[
            pltpu.VMEM((CH,), jnp.int32),
            pltpu.VMEM((CH,), jnp.int32),
            pltpu.VMEM((CH, VD), jnp.float32),
            pltpu.VMEM((CH, VD), jnp.float32),
            pltpu.VMEM((max(tail, 8),), jnp.int32),
            pltpu.VMEM((max(tail, 8), VD), jnp.float32),
            pltpu.VMEM_SHARED((NPAD, VD), jnp.float32),
            pltpu.SemaphoreType.DMA,
            pltpu.SemaphoreType.DMA,
        ],
    )
    def scat(v0, v1, v2, v3, v4, eidx, zeros,
             o0, o1, o2, o3, o4,
             idxA, idxB, valA, valB, idxT, valT, acc, semA, semB):
        c = lax.axis_index("c")
        s = lax.axis_index("s")
        base = c * EPC + s * W
        row0 = s * RPT
        ibufs = (idxA, idxB)
        vbufs = (valA, valB)
        sems = (semA, semB)
        vrefs = (v0, v1, v2, v3, v4)
        orefs = (o0, o1, o2, o3, o4)

        for p in range(5):
            v = vrefs[p]

            def lstart(g, k):
                b = base + g * CH
                pltpu.async_copy(eidx.at[pl.ds(b, CH)], ibufs[k], sems[k])
                pltpu.async_copy(v.at[pl.ds(b, CH), :], vbufs[k], sems[k])

            def do_chunk(g, k):
                b = base + g * CH
                pltpu.make_async_copy(eidx.at[pl.ds(b, CH)], ibufs[k],
                                      sems[k]).wait()
                pltpu.make_async_copy(v.at[pl.ds(b, CH), :], vbufs[k],
                                      sems[k]).wait()
                pltpu.sync_copy(vbufs[k], acc.at[ibufs[k]], add=True)

            pltpu.sync_copy(zeros.at[pl.ds(row0, RPT), :],
                            acc.at[pl.ds(row0, RPT), :])
            plsc.subcore_barrier()

            lstart(0, 0)

            def outer(g0, _):
                for b in range(2):
                    g = g0 + b
                    @pl.when(g < full)
                    def _():
                        @pl.when(g + 1 < full)
                        def _():
                            lstart(g + 1, (b + 1) % 2)
                        do_chunk(g, b)
                return _

            lax.fori_loop(0, (full + 1) // 2, lambda i, u: outer(2 * i, u),
                          None)
            if tail > 0:
                b = base + full * CH
                pltpu.sync_copy(eidx.at[pl.ds(b, tail)], idxT)
                pltpu.sync_copy(v.at[pl.ds(b, tail), :], valT)
                pltpu.sync_copy(valT, acc.at[idxT], add=True)

            plsc.subcore_barrier()
            pltpu.sync_copy(acc.at[pl.ds(row0, RPT), :],
                            orefs[p].at[c, pl.ds(row0, RPT), :])
            plsc.subcore_barrier()

    return scat


def _edge_body(hj, hi, W1a, W1b, w1c, b1, W2, b2, o0, o1, o2, o3, o4):
    hjh = hj[:, :H]
    hih = hi[:, :H]
    rx = hj[:, H:H + 1] - hi[:, H:H + 1]
    ry = hj[:, H + 1:H + 2] - hi[:, H + 1:H + 2]
    rz = hj[:, H + 2:H + 3] - hi[:, H + 2:H + 3]
    d2 = rx * rx + ry * ry + rz * rz + 1e-12
    dist = jnp.sqrt(d2)
    t = hih @ W1a[...] + hjh @ W1b[...] + dist * w1c[...] + b1[...]
    t = _silu(t)
    m = t @ W2[...] + b2[...]
    x = jnp.clip(dist / CUTOFF, 0.0, 1.0)
    w = 1.0 - 6.0 * x ** 5 + 15.0 * x ** 4 - 10.0 * x ** 3
    m = m * w
    inv = 1.0 / (dist + 1e-8)
    m32 = m[:, :VD]
    o0[...] = m[:, :VD]
    o1[...] = m[:, VD:]
    o2[...] = (rx * inv) * m32
    o3[...] = (ry * inv) * m32
    o4[...] = (rz * inv) * m32


def _const_spec(shape):
    return pl.BlockSpec(shape, lambda i: (0,) * len(shape))


def _edge_call(hj80, hi80, ew):
    EP = hj80.shape[0]
    W1, b1 = ew[0]
    W2, b2 = ew[1]
    especs = [
        pl.BlockSpec((BE, 80), lambda i: (i, 0)),
        pl.BlockSpec((BE, 80), lambda i: (i, 0)),
        _const_spec((H, H)), _const_spec((H, H)), _const_spec((1, H)),
        _const_spec((1, H)), _const_spec((H, H)), _const_spec((1, H)),
    ]
    ospec = pl.BlockSpec((BE, VD), lambda i: (i, 0))
    osh = jax.ShapeDtypeStruct((EP, VD), jnp.float32)
    return pl.pallas_call(
        _edge_body,
        grid=(EP // BE,),
        in_specs=especs,
        out_specs=[ospec] * 5,
        out_shape=[osh] * 5,
    )(hj80, hi80, W1[:H], W1[H:2 * H], W1[2 * H:2 * H + 1], b1[None, :],
      W2, b2[None, :])


def _node_body(h, V, p0, p1, p2, p3, p4,
               nW1, nb1, nW2, nb2,
               gW1, gb1, gW2, gb2,
               mW1a, mW1b, mb1, mW2, mb2,
               qW1, qb1, qW2, qb2,
               h_out, V_out):
    hmv = jnp.concatenate([p0[0] + p0[1], p1[0] + p1[1]], axis=1)
    t = _silu(hmv @ nW1[...] + nb1[...])
    hu = t @ nW2[...] + nb2[...]
    t = _silu(hmv @ gW1[...] + gb1[...])
    gate = t @ gW2[...] + gb2[...]
    sg = jax.nn.sigmoid(gate)
    Vn = V[...] + jnp.concatenate(
        [(p2[0] + p2[1]) * sg, (p3[0] + p3[1]) * sg, (p4[0] + p4[1]) * sg],
        axis=1)
    hn = h[...] + hu
    Vx = Vn[:, :VD]
    Vy = Vn[:, VD:2 * VD]
    Vz = Vn[:, 2 * VD:]
    vnorm = jnp.sqrt(Vx * Vx + Vy * Vy + Vz * Vz + 1e-12)
    t = _silu(hn @ mW1a[...] + vnorm @ mW1b[...] + mb1[...])
    hn = hn + (t @ mW2[...] + mb2[...])
    t = _silu(hn @ qW1[...] + qb1[...])
    g2 = t @ qW2[...] + qb2[...]
    sq = jax.nn.sigmoid(g2)
    V_out[...] = Vn * jnp.concatenate([sq, sq, sq], axis=1)
    h_out[...] = hn


def _node_call(hp, Vp, parts, mpp, mixp):
    NP = hp.shape[0]
    nW1, nb1 = mpp['node'][0]
    nW2, nb2 = mpp['node'][1]
    gW1, gb1 = mpp['vgate'][0]
    gW2, gb2 = mpp['vgate'][1]
    mW1, mb1 = mixp['norm'][0]
    mW2, mb2 = mixp['norm'][1]
    qW1, qb1 = mixp['gate'][0]
    qW2, qb2 = mixp['gate'][1]
    dspec = lambda w: pl.BlockSpec((BN, w), lambda i: (i, 0))
    pspec = pl.BlockSpec((NC, BN, VD), lambda i: (0, i, 0))
    specs = [dspec(H), dspec(96)] + [pspec] * 5 + [
        _const_spec((H, H)), _const_spec((1, H)), _const_spec((H, H)), _const_spec((1, H)),
        _const_spec((H, H)), _const_spec((1, H)), _const_spec((H, VD)), _const_spec((1, VD)),
        _const_spec((H, H)), _const_spec((VD, H)), _const_spec((1, H)),
        _const_spec((H, H)), _const_spec((1, H)),
        _const_spec((H, H)), _const_spec((1, H)), _const_spec((H, VD)), _const_spec((1, VD))]
    return pl.pallas_call(
        _node_body,
        grid=(NP // BN,),
        in_specs=specs,
        out_specs=[dspec(H), dspec(96)],
        out_shape=[jax.ShapeDtypeStruct((NP, H), jnp.float32),
                   jax.ShapeDtypeStruct((NP, 96), jnp.float32)],
    )(hp, Vp, *parts,
      nW1, nb1[None, :], nW2, nb2[None, :],
      gW1, gb1[None, :], gW2, gb2[None, :],
      mW1[:H], mW1[H:], mb1[None, :], mW2, mb2[None, :],
      qW1, qb1[None, :], qW2, qb2[None, :])


def _readout_body(h, bb, oW1, ob1, oW2, ob2, atom_out, mol_out, acc):
    i = pl.program_id(0)
    t = _silu(h[...] @ oW1[...] + ob1[...])
    a8 = t @ oW2[...] + ob2[...]
    atom_out[...] = a8[:, 0:1]
    lanes = jax.lax.broadcasted_iota(jnp.int32, (BN, NG), 1)
    oh = (bb[...] == lanes).astype(jnp.float32)
    stat = jnp.concatenate(
        [a8[:, 0:1], jnp.ones((BN, 1), jnp.float32),
         jnp.zeros((BN, 6), jnp.float32)], axis=1)
    part = jax.lax.dot_general(oh, stat, (((0,), (0,)), ((), ())))

    @pl.when(i == 0)
    def _():
        acc[...] = part

    @pl.when(i != 0)
    def _():
        acc[...] = acc[...] + part

    @pl.when(i == pl.num_programs(0) - 1)
    def _():
        a = acc[...]
        mol_out[...] = a[:, 0:1] / jnp.clip(a[:, 1:2], 1.0, None)


def _readout_call(hp, bbp, ow):
    NP = hp.shape[0]
    oW1, ob1 = ow[0]
    oW2, ob2 = ow[1]
    oW2p = jnp.pad(oW2, ((0, 0), (0, 7)))
    ob2p = jnp.pad(ob2, (0, 7))[None, :]
    return pl.pallas_call(
        _readout_body,
        grid=(NP // BN,),
        in_specs=[pl.BlockSpec((BN, H), lambda i: (i, 0)),
                  pl.BlockSpec((BN, 1), lambda i: (i, 0)),
                  _const_spec((H, H)), _const_spec((1, H)),
                  _const_spec((H, 8)), _const_spec((1, 8))],
        out_specs=[pl.BlockSpec((BN, 1), lambda i: (i, 0)),
                   pl.BlockSpec((NG, 1), lambda i: (0, 0))],
        out_shape=[jax.ShapeDtypeStruct((NP, 1), jnp.float32),
                   jax.ShapeDtypeStruct((NG, 1), jnp.float32)],
        scratch_shapes=[pltpu.VMEM((NG, 8), jnp.float32)],
    )(hp, bbp, oW1, ob1[None, :], oW2p, ob2p)


def kernel(z, pos, edge_index, batch, params):
    N = z.shape[0]
    E = edge_index.shape[1]
    NP = -(-N // BN) * BN
    EP = -(-E // BE) * BE

    src = edge_index[0]
    dst = edge_index[1].astype(jnp.int32)
    posp = jnp.pad(pos, ((0, 0), (0, 13)))

    NZ = -(-N // (NW * CH)) * (NW * CH)
    zp = jnp.pad(z.astype(jnp.int32), (0, NZ - N))
    embp = jnp.pad(params['emb'], ((0, 0), (0, 16)))
    h = _make_gather(embp.shape[0], NZ, 80)(embp, zp)[:N, :H]

    V = jnp.zeros((N, 96), jnp.float32)
    eidx2 = edge_index.astype(jnp.int32).reshape(-1)
    E2 = -(-2 * E // (NW * CH)) * (NW * CH)
    eidx2 = jnp.pad(eidx2, (0, E2 - 2 * E))
    dstp = jnp.pad(dst, (0, EP - E), constant_values=NP - 1)
    zeros_np = jnp.zeros((NP, VD), jnp.float32)

    gather_T = _make_gather(N, E2, 80)
    scatter = _make_scatter(EP, NP)

    for mpp, mixp in zip(params['mp'], params['mix']):
        T = jnp.concatenate([h, posp], axis=1)
        rows = gather_T(T, eidx2)
        hj80 = rows[:E]
        hi80 = rows[E:2 * E]
        if EP != E:
            hj80 = jnp.pad(hj80, ((0, EP - E), (0, 0)))
            hi80 = jnp.pad(hi80, ((0, EP - E), (0, 0)))
        eouts = _edge_call(hj80, hi80, mpp['edge'])
        parts = scatter(*eouts, dstp, zeros_np)
        hp, Vp = _node_call(
            jnp.pad(h, ((0, NP - N), (0, 0))),
            jnp.pad(V, ((0, NP - N), (0, 0))),
            parts, mpp, mixp)
        h, V = hp[:N], Vp[:N]

    bbp = jnp.pad(batch.astype(jnp.int32), (0, NP - N),
                  constant_values=NG)[:, None]
    atomp, mol = _readout_call(jnp.pad(h, ((0, NP - N), (0, 0))), bbp,
                               params['out'])
    return (mol, atomp[:N])


# speedup vs baseline: 17.4566x; 1.1026x over previous
import functools

import jax
import jax.numpy as jnp
from jax import lax
from jax.experimental import pallas as pl
from jax.experimental.pallas import tpu as pltpu
from jax.experimental.pallas import tpu_sc as plsc

H = 64
VD = 32
NG = 64
CUTOFF = 5.0

BE = 4000
BN = 1024

NC = 2
NS = 16
NW = NC * NS
CH = 128


def _silu(x):
    return x * jax.nn.sigmoid(x)


def _mesh():
    return plsc.VectorSubcoreMesh(core_axis_name="c", subcore_axis_name="s",
                                  num_cores=NC, num_subcores=NS)


@functools.lru_cache(maxsize=None)
def _make_gather(TR, NI, D):
    W = NI // NW
    full, tail = divmod(W, CH)

    @functools.partial(
        pl.kernel,
        out_type=jax.ShapeDtypeStruct((NI, D), jnp.float32),
        mesh=_mesh(),
        compiler_params=pltpu.CompilerParams(use_tc_tiling_on_sc=False),
        scratch_types=[
            pltpu.VMEM((CH,), jnp.int32),
            pltpu.VMEM((CH,), jnp.int32),
            pltpu.VMEM((CH, D), jnp.float32),
            pltpu.SemaphoreType.DMA,
            pltpu.SemaphoreType.DMA,
            pltpu.SemaphoreType.DMA,
        ],
    )
    def gath(tab, eidx, out, idx0, idx1, rows, semi0, semi1, semg):
        c = lax.axis_index("c")
        s = lax.axis_index("s")
        base = (s * NC + c) * W
        ibufs = (idx0, idx1)
        isems = (semi0, semi1)

        def istart(g, k, n):
            pltpu.async_copy(eidx.at[pl.ds(base + g * CH, n)],
                             ibufs[k].at[pl.ds(0, n)], isems[k])

        def iwait(g, k, n):
            pltpu.make_async_copy(eidx.at[pl.ds(base + g * CH, n)],
                                  ibufs[k].at[pl.ds(0, n)], isems[k]).wait()

        def do_chunk(g, k, n):
            iwait(g, k, n)
            pltpu.async_copy(tab.at[ibufs[k].at[pl.ds(0, n)]],
                             rows.at[pl.ds(0, n), :], semg).wait()
            pltpu.sync_copy(rows.at[pl.ds(0, n), :],
                            out.at[pl.ds(base + g * CH, n), :])

        if full > 0:
            istart(0, 0, CH)

            def outer(g0, _):
                for b in range(2):
                    g = g0 + b
                    @pl.when(g < full)
                    def _():
                        nxt = g + 1
                        @pl.when(nxt < full)
                        def _():
                            istart(nxt, (b + 1) % 2, CH)
                        @pl.when(jnp.logical_and(nxt == full, tail > 0))
                        def _():
                            istart(nxt, (b + 1) % 2, max(tail, 1))
                        do_chunk(g, b, CH)
                return _

            lax.fori_loop(0, (full + 1) // 2, lambda i, u: outer(2 * i, u),
                          None)
            if tail > 0:
                do_chunk(full, full % 2, tail)
        else:
            if tail > 0:
                istart(0, 0, tail)
                do_chunk(0, 0, tail)

    return gath


@functools.lru_cache(maxsize=None)
def _make_scatter(EP, NPAD):
    W = EP // NW
    full, tail = divmod(W, CH)
    RPT = NPAD // NS
    EPC = EP // NC

    odt = [jax.ShapeDtypeStruct((NC, NPAD, VD), jnp.float32)
           for _ in range(5)]

    @functools.partial(
        pl.kernel,
        out_type=odt,
        mesh=_mesh(),
        compiler_params=pltpu.CompilerParams(use_tc_tiling_on_sc=False),
        scratch_types=[
            pltpu.VMEM((CH,), jnp.int32),
            pltpu.VMEM((CH,), jnp.int32),
            pltpu.VMEM((CH, VD), jnp.float32),
            pltpu.VMEM((CH, VD), jnp.float32),
            pltpu.VMEM((max(tail, 8),), jnp.int32),
            pltpu.VMEM((max(tail, 8), VD), jnp.float32),
            pltpu.VMEM_SHARED((NPAD, VD), jnp.float32),
            pltpu.SemaphoreType.DMA,
            pltpu.SemaphoreType.DMA,
        ],
    )
    def scat(v0, v1, v2, v3, v4, eidx, zeros,
             o0, o1, o2, o3, o4,
             idxA, idxB, valA, valB, idxT, valT, acc, semA, semB):
        c = lax.axis_index("c")
        s = lax.axis_index("s")
        base = c * EPC + s * W
        row0 = s * RPT
        ibufs = (idxA, idxB)
        vbufs = (valA, valB)
        sems = (semA, semB)
        vrefs = (v0, v1, v2, v3, v4)
        orefs = (o0, o1, o2, o3, o4)

        for p in range(5):
            v = vrefs[p]

            def lstart(g, k):
                b = base + g * CH
                pltpu.async_copy(eidx.at[pl.ds(b, CH)], ibufs[k], sems[k])
                pltpu.async_copy(v.at[pl.ds(b, CH), :], vbufs[k], sems[k])

            def do_chunk(g, k):
                b = base + g * CH
                pltpu.make_async_copy(eidx.at[pl.ds(b, CH)], ibufs[k],
                                      sems[k]).wait()
                pltpu.make_async_copy(v.at[pl.ds(b, CH), :], vbufs[k],
                                      sems[k]).wait()
                pltpu.sync_copy(vbufs[k], acc.at[ibufs[k]], add=True)

            pltpu.sync_copy(zeros.at[pl.ds(row0, RPT), :],
                            acc.at[pl.ds(row0, RPT), :])
            plsc.subcore_barrier()

            lstart(0, 0)

            def outer(g0, _):
                for b in range(2):
                    g = g0 + b
                    @pl.when(g < full)
                    def _():
                        @pl.when(g + 1 < full)
                        def _():
                            lstart(g + 1, (b + 1) % 2)
                        do_chunk(g, b)
                return _

            lax.fori_loop(0, (full + 1) // 2, lambda i, u: outer(2 * i, u),
                          None)
            if tail > 0:
                b = base + full * CH
                pltpu.sync_copy(eidx.at[pl.ds(b, tail)], idxT)
                pltpu.sync_copy(v.at[pl.ds(b, tail), :], valT)
                pltpu.sync_copy(valT, acc.at[idxT], add=True)

            plsc.subcore_barrier()
            pltpu.sync_copy(acc.at[pl.ds(row0, RPT), :],
                            orefs[p].at[c, pl.ds(row0, RPT), :])
            plsc.subcore_barrier()

    return scat


def _edge_body(hj, hi, W1a, W1b, w1c, b1, W2, b2, o0, o1, o2, o3, o4):
    hjh = hj[:, :H]
    hih = hi[:, :H]
    rx = hj[:, H:H + 1] - hi[:, H:H + 1]
    ry = hj[:, H + 1:H + 2] - hi[:, H + 1:H + 2]
    rz = hj[:, H + 2:H + 3] - hi[:, H + 2:H + 3]
    d2 = rx * rx + ry * ry + rz * rz + 1e-12
    dist = jnp.sqrt(d2)
    t = hih @ W1a[...] + hjh @ W1b[...] + dist * w1c[...] + b1[...]
    t = _silu(t)
    m = t @ W2[...] + b2[...]
    x = jnp.clip(dist / CUTOFF, 0.0, 1.0)
    w = 1.0 - 6.0 * x ** 5 + 15.0 * x ** 4 - 10.0 * x ** 3
    m = m * w
    inv = 1.0 / (dist + 1e-8)
    m32 = m[:, :VD]
    o0[...] = m[:, :VD]
    o1[...] = m[:, VD:]
    o2[...] = (rx * inv) * m32
    o3[...] = (ry * inv) * m32
    o4[...] = (rz * inv) * m32


def _const_spec(shape):
    return pl.BlockSpec(shape, lambda i: (0,) * len(shape))


def _edge_call(rows, E, ew):
    nb = E // BE
    W1, b1 = ew[0]
    W2, b2 = ew[1]
    especs = [
        pl.BlockSpec((BE, 80), lambda i: (i, 0)),
        pl.BlockSpec((BE, 80), lambda i: (i + nb, 0)),
        _const_spec((H, H)), _const_spec((H, H)), _const_spec((1, H)),
        _const_spec((1, H)), _const_spec((H, H)), _const_spec((1, H)),
    ]
    ospec = pl.BlockSpec((BE, VD), lambda i: (i, 0))
    osh = jax.ShapeDtypeStruct((E, VD), jnp.float32)
    return pl.pallas_call(
        _edge_body,
        grid=(nb,),
        in_specs=especs,
        out_specs=[ospec] * 5,
        out_shape=[osh] * 5,
    )(rows, rows, W1[:H], W1[H:2 * H], W1[2 * H:2 * H + 1], b1[None, :],
      W2, b2[None, :])


def _node_body(h, V, p0, p1, p2, p3, p4,
               nW1, nb1, nW2, nb2,
               gW1, gb1, gW2, gb2,
               mW1a, mW1b, mb1, mW2, mb2,
               qW1, qb1, qW2, qb2,
               h_out, V_out):
    hmv = jnp.concatenate([p0[0] + p0[1], p1[0] + p1[1]], axis=1)
    t = _silu(hmv @ nW1[...] + nb1[...])
    hu = t @ nW2[...] + nb2[...]
    t = _silu(hmv @ gW1[...] + gb1[...])
    gate = t @ gW2[...] + gb2[...]
    sg = jax.nn.sigmoid(gate)
    Vn = V[...] + jnp.concatenate(
        [(p2[0] + p2[1]) * sg, (p3[0] + p3[1]) * sg, (p4[0] + p4[1]) * sg],
        axis=1)
    hn = h[...] + hu
    Vx = Vn[:, :VD]
    Vy = Vn[:, VD:2 * VD]
    Vz = Vn[:, 2 * VD:]
    vnorm = jnp.sqrt(Vx * Vx + Vy * Vy + Vz * Vz + 1e-12)
    t = _silu(hn @ mW1a[...] + vnorm @ mW1b[...] + mb1[...])
    hn = hn + (t @ mW2[...] + mb2[...])
    t = _silu(hn @ qW1[...] + qb1[...])
    g2 = t @ qW2[...] + qb2[...]
    sq = jax.nn.sigmoid(g2)
    V_out[...] = Vn * jnp.concatenate([sq, sq, sq], axis=1)
    h_out[...] = hn


def _node_call(hp, Vp, parts, mpp, mixp):
    NP = hp.shape[0]
    nW1, nb1 = mpp['node'][0]
    nW2, nb2 = mpp['node'][1]
    gW1, gb1 = mpp['vgate'][0]
    gW2, gb2 = mpp['vgate'][1]
    mW1, mb1 = mixp['norm'][0]
    mW2, mb2 = mixp['norm'][1]
    qW1, qb1 = mixp['gate'][0]
    qW2, qb2 = mixp['gate'][1]
    dspec = lambda w: pl.BlockSpec((BN, w), lambda i: (i, 0))
    pspec = pl.BlockSpec((NC, BN, VD), lambda i: (0, i, 0))
    specs = [dspec(H), dspec(96)] + [pspec] * 5 + [
        _const_spec((H, H)), _const_spec((1, H)), _const_spec((H, H)), _const_spec((1, H)),
        _const_spec((H, H)), _const_spec((1, H)), _const_spec((H, VD)), _const_spec((1, VD)),
        _const_spec((H, H)), _const_spec((VD, H)), _const_spec((1, H)),
        _const_spec((H, H)), _const_spec((1, H)),
        _const_spec((H, H)), _const_spec((1, H)), _const_spec((H, VD)), _const_spec((1, VD))]
    return pl.pallas_call(
        _node_body,
        grid=(NP // BN,),
        in_specs=specs,
        out_specs=[dspec(H), dspec(96)],
        out_shape=[jax.ShapeDtypeStruct((NP, H), jnp.float32),
                   jax.ShapeDtypeStruct((NP, 96), jnp.float32)],
    )(hp, Vp, *parts,
      nW1, nb1[None, :], nW2, nb2[None, :],
      gW1, gb1[None, :], gW2, gb2[None, :],
      mW1[:H], mW1[H:], mb1[None, :], mW2, mb2[None, :],
      qW1, qb1[None, :], qW2, qb2[None, :])


def _readout_body(h, bb, oW1, ob1, oW2, ob2, atom_out, mol_out, acc):
    i = pl.program_id(0)
    t = _silu(h[...] @ oW1[...] + ob1[...])
    a8 = t @ oW2[...] + ob2[...]
    atom_out[...] = a8[:, 0:1]
    lanes = jax.lax.broadcasted_iota(jnp.int32, (BN, NG), 1)
    oh = (bb[...] == lanes).astype(jnp.float32)
    stat = jnp.concatenate(
        [a8[:, 0:1], jnp.ones((BN, 1), jnp.float32),
         jnp.zeros((BN, 6), jnp.float32)], axis=1)
    part = jax.lax.dot_general(oh, stat, (((0,), (0,)), ((), ())))

    @pl.when(i == 0)
    def _():
        acc[...] = part

    @pl.when(i != 0)
    def _():
        acc[...] = acc[...] + part

    @pl.when(i == pl.num_programs(0) - 1)
    def _():
        a = acc[...]
        mol_out[...] = a[:, 0:1] / jnp.clip(a[:, 1:2], 1.0, None)


def _readout_call(hp, bbp, ow):
    NP = hp.shape[0]
    oW1, ob1 = ow[0]
    oW2, ob2 = ow[1]
    oW2p = jnp.pad(oW2, ((0, 0), (0, 7)))
    ob2p = jnp.pad(ob2, (0, 7))[None, :]
    return pl.pallas_call(
        _readout_body,
        grid=(NP // BN,),
        in_specs=[pl.BlockSpec((BN, H), lambda i: (i, 0)),
                  pl.BlockSpec((BN, 1), lambda i: (i, 0)),
                  _const_spec((H, H)), _const_spec((1, H)),
                  _const_spec((H, 8)), _const_spec((1, 8))],
        out_specs=[pl.BlockSpec((BN, 1), lambda i: (i, 0)),
                   pl.BlockSpec((NG, 1), lambda i: (0, 0))],
        out_shape=[jax.ShapeDtypeStruct((NP, 1), jnp.float32),
                   jax.ShapeDtypeStruct((NG, 1), jnp.float32)],
        scratch_shapes=[pltpu.VMEM((NG, 8), jnp.float32)],
    )(hp, bbp, oW1, ob1[None, :], oW2p, ob2p)


def kernel(z, pos, edge_index, batch, params):
    N = z.shape[0]
    E = edge_index.shape[1]
    NP = -(-N // BN) * BN
    EP = -(-E // BE) * BE

    dst = edge_index[1].astype(jnp.int32)
    posp = jnp.pad(pos, ((0, NP - N), (0, 13)))

    NZ = -(-N // (NW * CH)) * (NW * CH)
    zp = jnp.pad(z.astype(jnp.int32), (0, NZ - N))
    embp = jnp.pad(params['emb'], ((0, 0), (0, 16)))
    h = _make_gather(embp.shape[0], NZ, 80)(embp, zp)[:NP, :H]

    V = jnp.zeros((NP, 96), jnp.float32)
    eidx2 = edge_index.astype(jnp.int32).reshape(-1)
    E2 = -(-2 * E // (NW * CH)) * (NW * CH)
    eidx2 = jnp.pad(eidx2, (0, E2 - 2 * E))
    dstp = jnp.pad(dst, (0, EP - E), constant_values=NP - 1)
    zeros_np = jnp.zeros((NP, VD), jnp.float32)

    gather_T = _make_gather(NP, E2, 80)
    scatter = _make_scatter(EP, NP)

    for mpp, mixp in zip(params['mp'], params['mix']):
        T = jnp.concatenate([h, posp], axis=1)
        rows = gather_T(T, eidx2)
        eouts = _edge_call(rows, E, mpp['edge'])
        parts = scatter(*eouts, dstp, zeros_np)
        h, V = _node_call(h, V, parts, mpp, mixp)

    bbp = jnp.pad(batch.astype(jnp.int32), (0, NP - N),
                  constant_values=NG)[:, None]
    atomp, mol = _readout_call(h, bbp, params['out'])
    return (mol, atomp[:N])


# speedup vs baseline: 27.5433x; 1.5778x over previous
import functools

import jax
import jax.numpy as jnp
from jax import lax
from jax.experimental import pallas as pl
from jax.experimental.pallas import tpu as pltpu
from jax.experimental.pallas import tpu_sc as plsc

H = 64
VD = 32
NG = 64
CUTOFF = 5.0

BE = 4000
BN = 1024

NC = 2
NS = 16
NW = NC * NS
CH = 128
TW = 128


def _silu(x):
    return x * jax.nn.sigmoid(x)


def _mesh():
    return plsc.VectorSubcoreMesh(core_axis_name="c", subcore_axis_name="s",
                                  num_cores=NC, num_subcores=NS)


@functools.lru_cache(maxsize=None)
def _make_gather(TR, NI, D):
    W = NI // NW
    full, tail = divmod(W, CH)

    @functools.partial(
        pl.kernel,
        out_type=jax.ShapeDtypeStruct((NI, D), jnp.float32),
        mesh=_mesh(),
        compiler_params=pltpu.CompilerParams(use_tc_tiling_on_sc=False),
        scratch_types=[
            pltpu.VMEM((CH,), jnp.int32),
            pltpu.VMEM((CH,), jnp.int32),
            pltpu.VMEM((CH, D), jnp.float32),
            pltpu.SemaphoreType.DMA,
            pltpu.SemaphoreType.DMA,
            pltpu.SemaphoreType.DMA,
        ],
    )
    def gath(tab, eidx, out, idx0, idx1, rows, semi0, semi1, semg):
        c = lax.axis_index("c")
        s = lax.axis_index("s")
        base = (s * NC + c) * W
        ibufs = (idx0, idx1)
        isems = (semi0, semi1)

        def istart(g, k, n):
            pltpu.async_copy(eidx.at[pl.ds(base + g * CH, n)],
                             ibufs[k].at[pl.ds(0, n)], isems[k])

        def iwait(g, k, n):
            pltpu.make_async_copy(eidx.at[pl.ds(base + g * CH, n)],
                                  ibufs[k].at[pl.ds(0, n)], isems[k]).wait()

        def do_chunk(g, k, n):
            iwait(g, k, n)
            pltpu.async_copy(tab.at[ibufs[k].at[pl.ds(0, n)]],
                             rows.at[pl.ds(0, n), :], semg).wait()
            pltpu.sync_copy(rows.at[pl.ds(0, n), :],
                            out.at[pl.ds(base + g * CH, n), :])

        if full > 0:
            istart(0, 0, CH)

            def outer(g0, _):
                for b in range(2):
                    g = g0 + b
                    @pl.when(g < full)
                    def _():
                        nxt = g + 1
                        @pl.when(nxt < full)
                        def _():
                            istart(nxt, (b + 1) % 2, CH)
                        @pl.when(jnp.logical_and(nxt == full, tail > 0))
                        def _():
                            istart(nxt, (b + 1) % 2, max(tail, 1))
                        do_chunk(g, b, CH)
                return _

            lax.fori_loop(0, (full + 1) // 2, lambda i, u: outer(2 * i, u),
                          None)
            if tail > 0:
                do_chunk(full, full % 2, tail)
        else:
            if tail > 0:
                istart(0, 0, tail)
                do_chunk(0, 0, tail)

    return gath


@functools.lru_cache(maxsize=None)
def _make_scatter(EP, NPAD):
    W = EP // NW
    full, tail = divmod(W, CH)
    RPT = NPAD // NS
    EPC = EP // NC

    odt = [jax.ShapeDtypeStruct((NC, NPAD, 128), jnp.float32),
           jax.ShapeDtypeStruct((NC, NPAD, VD), jnp.float32)]

    @functools.partial(
        pl.kernel,
        out_type=odt,
        mesh=_mesh(),
        compiler_params=pltpu.CompilerParams(use_tc_tiling_on_sc=False),
        scratch_types=[
            pltpu.VMEM((CH,), jnp.int32),
            pltpu.VMEM((CH,), jnp.int32),
            pltpu.VMEM((CH, VD), jnp.float32),
            pltpu.VMEM((CH, VD), jnp.float32),
            pltpu.VMEM((max(tail, 8),), jnp.int32),
            pltpu.VMEM((max(tail, 8), VD), jnp.float32),
            pltpu.VMEM_SHARED((NPAD, VD), jnp.float32),
            pltpu.SemaphoreType.DMA,
            pltpu.SemaphoreType.DMA,
        ],
    )
    def scat(va, vb, eidx, zeros, oa, ob,
             idxA, idxB, valA, valB, idxT, valT, acc, semA, semB):
        c = lax.axis_index("c")
        s = lax.axis_index("s")
        base = c * EPC + s * W
        row0 = s * RPT
        ibufs = (idxA, idxB)
        vbufs = (valA, valB)
        sems = (semA, semB)

        for p in range(5):
            v = va if p < 4 else vb
            off = (p * VD) if p < 4 else 0

            def lstart(g, k):
                b = base + g * CH
                pltpu.async_copy(eidx.at[pl.ds(b, CH)], ibufs[k], sems[k])
                pltpu.async_copy(v.at[pl.ds(b, CH), pl.ds(off, VD)],
                                 vbufs[k], sems[k])

            def do_chunk(g, k):
                b = base + g * CH
                pltpu.make_async_copy(eidx.at[pl.ds(b, CH)], ibufs[k],
                                      sems[k]).wait()
                pltpu.make_async_copy(v.at[pl.ds(b, CH), pl.ds(off, VD)],
                                      vbufs[k], sems[k]).wait()
                pltpu.sync_copy(vbufs[k], acc.at[ibufs[k]], add=True)

            pltpu.sync_copy(zeros.at[pl.ds(row0, RPT), :],
                            acc.at[pl.ds(row0, RPT), :])
            plsc.subcore_barrier()

            lstart(0, 0)

            def outer(g0, _):
                for b in range(2):
                    g = g0 + b
                    @pl.when(g < full)
                    def _():
                        @pl.when(g + 1 < full)
                        def _():
                            lstart(g + 1, (b + 1) % 2)
                        do_chunk(g, b)
                return _

            lax.fori_loop(0, (full + 1) // 2, lambda i, u: outer(2 * i, u),
                          None)
            if tail > 0:
                b = base + full * CH
                pltpu.sync_copy(eidx.at[pl.ds(b, tail)], idxT)
                pltpu.sync_copy(v.at[pl.ds(b, tail), pl.ds(off, VD)], valT)
                pltpu.sync_copy(valT, acc.at[idxT], add=True)

            plsc.subcore_barrier()
            if p < 4:
                pltpu.sync_copy(acc.at[pl.ds(row0, RPT), :],
                                oa.at[c, pl.ds(row0, RPT), pl.ds(off, VD)])
            else:
                pltpu.sync_copy(acc.at[pl.ds(row0, RPT), :],
                                ob.at[c, pl.ds(row0, RPT), :])
            plsc.subcore_barrier()

    return scat


def _edge_body(hj, hi, W1a, W1b, w1c, b1, W2, b2, oa, ob):
    hjh = hj[:, :H]
    hih = hi[:, :H]
    rx = hj[:, H:H + 1] - hi[:, H:H + 1]
    ry = hj[:, H + 1:H + 2] - hi[:, H + 1:H + 2]
    rz = hj[:, H + 2:H + 3] - hi[:, H + 2:H + 3]
    d2 = rx * rx + ry * ry + rz * rz + 1e-12
    dist = jnp.sqrt(d2)
    t = hih @ W1a[...] + hjh @ W1b[...] + dist * w1c[...] + b1[...]
    t = _silu(t)
    m = t @ W2[...] + b2[...]
    x = jnp.clip(dist / CUTOFF, 0.0, 1.0)
    w = 1.0 - 6.0 * x ** 5 + 15.0 * x ** 4 - 10.0 * x ** 3
    m = m * w
    inv = 1.0 / (dist + 1e-8)
    m32 = m[:, :VD]
    oa[...] = jnp.concatenate([m, (rx * inv) * m32, (ry * inv) * m32], axis=1)
    ob[...] = jnp.concatenate(
        [(rz * inv) * m32, jnp.zeros((m.shape[0], 96), jnp.float32)], axis=1)


def _const_spec(shape):
    return pl.BlockSpec(shape, lambda i: (0,) * len(shape))


def _edge_call(rows, E, ew):
    nb = E // BE
    W1, b1 = ew[0]
    W2, b2 = ew[1]
    especs = [
        pl.BlockSpec((BE, TW), lambda i: (i, 0)),
        pl.BlockSpec((BE, TW), lambda i: (i + nb, 0)),
        _const_spec((H, H)), _const_spec((H, H)), _const_spec((1, H)),
        _const_spec((1, H)), _const_spec((H, H)), _const_spec((1, H)),
    ]
    ospec = pl.BlockSpec((BE, 128), lambda i: (i, 0))
    osh = jax.ShapeDtypeStruct((E, 128), jnp.float32)
    return pl.pallas_call(
        _edge_body,
        grid=(nb,),
        in_specs=especs,
        out_specs=[ospec, ospec],
        out_shape=[osh, osh],
    )(rows, rows, W1[:H], W1[H:2 * H], W1[2 * H:2 * H + 1], b1[None, :],
      W2, b2[None, :])


def _node_body(T, V, pa, pb,
               nW1, nb1, nW2, nb2,
               gW1, gb1, gW2, gb2,
               mW1a, mW1b, mb1, mW2, mb2,
               qW1, qb1, qW2, qb2,
               T_out, V_out):
    Tv = T[...]
    h = Tv[:, :H]
    hmv = pa[0, :, :H] + pa[1, :, :H]
    vmx = pa[0, :, H:H + VD] + pa[1, :, H:H + VD]
    vmy = pa[0, :, H + VD:] + pa[1, :, H + VD:]
    vmz = pb[0] + pb[1]
    t = _silu(hmv @ nW1[...] + nb1[...])
    hu = t @ nW2[...] + nb2[...]
    t = _silu(hmv @ gW1[...] + gb1[...])
    gate = t @ gW2[...] + gb2[...]
    sg = jax.nn.sigmoid(gate)
    Vn = V[...] + jnp.concatenate([vmx * sg, vmy * sg, vmz * sg], axis=1)
    hn = h + hu
    Vx = Vn[:, :VD]
    Vy = Vn[:, VD:2 * VD]
    Vz = Vn[:, 2 * VD:]
    vnorm = jnp.sqrt(Vx * Vx + Vy * Vy + Vz * Vz + 1e-12)
    t = _silu(hn @ mW1a[...] + vnorm @ mW1b[...] + mb1[...])
    hn = hn + (t @ mW2[...] + mb2[...])
    t = _silu(hn @ qW1[...] + qb1[...])
    g2 = t @ qW2[...] + qb2[...]
    sq = jax.nn.sigmoid(g2)
    V_out[...] = Vn * jnp.concatenate([sq, sq, sq], axis=1)
    T_out[...] = jnp.concatenate([hn, Tv[:, H:]], axis=1)


def _node_call(Tp, Vp, parts, mpp, mixp):
    NP = Tp.shape[0]
    nW1, nb1 = mpp['node'][0]
    nW2, nb2 = mpp['node'][1]
    gW1, gb1 = mpp['vgate'][0]
    gW2, gb2 = mpp['vgate'][1]
    mW1, mb1 = mixp['norm'][0]
    mW2, mb2 = mixp['norm'][1]
    qW1, qb1 = mixp['gate'][0]
    qW2, qb2 = mixp['gate'][1]
    dspec = lambda w: pl.BlockSpec((BN, w), lambda i: (i, 0))
    paspec = pl.BlockSpec((NC, BN, 128), lambda i: (0, i, 0))
    pbspec = pl.BlockSpec((NC, BN, VD), lambda i: (0, i, 0))
    specs = [dspec(TW), dspec(96), paspec, pbspec] + [
        _const_spec((H, H)), _const_spec((1, H)), _const_spec((H, H)), _const_spec((1, H)),
        _const_spec((H, H)), _const_spec((1, H)), _const_spec((H, VD)), _const_spec((1, VD)),
        _const_spec((H, H)), _const_spec((VD, H)), _const_spec((1, H)),
        _const_spec((H, H)), _const_spec((1, H)),
        _const_spec((H, H)), _const_spec((1, H)), _const_spec((H, VD)), _const_spec((1, VD))]
    return pl.pallas_call(
        _node_body,
        grid=(NP // BN,),
        in_specs=specs,
        out_specs=[dspec(TW), dspec(96)],
        out_shape=[jax.ShapeDtypeStruct((NP, TW), jnp.float32),
                   jax.ShapeDtypeStruct((NP, 96), jnp.float32)],
    )(Tp, Vp, *parts,
      nW1, nb1[None, :], nW2, nb2[None, :],
      gW1, gb1[None, :], gW2, gb2[None, :],
      mW1[:H], mW1[H:], mb1[None, :], mW2, mb2[None, :],
      qW1, qb1[None, :], qW2, qb2[None, :])


def _readout_body(T, bb, oW1, ob1, oW2, ob2, atom_out, mol_out, acc):
    i = pl.program_id(0)
    t = _silu(T[:, :H] @ oW1[...] + ob1[...])
    a8 = t @ oW2[...] + ob2[...]
    atom_out[...] = a8[:, 0:1]
    lanes = jax.lax.broadcasted_iota(jnp.int32, (BN, NG), 1)
    oh = (bb[...] == lanes).astype(jnp.float32)
    stat = jnp.concatenate(
        [a8[:, 0:1], jnp.ones((BN, 1), jnp.float32),
         jnp.zeros((BN, 6), jnp.float32)], axis=1)
    part = jax.lax.dot_general(oh, stat, (((0,), (0,)), ((), ())))

    @pl.when(i == 0)
    def _():
        acc[...] = part

    @pl.when(i != 0)
    def _():
        acc[...] = acc[...] + part

    @pl.when(i == pl.num_programs(0) - 1)
    def _():
        a = acc[...]
        mol_out[...] = a[:, 0:1] / jnp.clip(a[:, 1:2], 1.0, None)


def _readout_call(Tp, bbp, ow):
    NP = Tp.shape[0]
    oW1, ob1 = ow[0]
    oW2, ob2 = ow[1]
    oW2p = jnp.pad(oW2, ((0, 0), (0, 7)))
    ob2p = jnp.pad(ob2, (0, 7))[None, :]
    return pl.pallas_call(
        _readout_body,
        grid=(NP // BN,),
        in_specs=[pl.BlockSpec((BN, TW), lambda i: (i, 0)),
                  pl.BlockSpec((BN, 1), lambda i: (i, 0)),
                  _const_spec((H, H)), _const_spec((1, H)),
                  _const_spec((H, 8)), _const_spec((1, 8))],
        out_specs=[pl.BlockSpec((BN, 1), lambda i: (i, 0)),
                   pl.BlockSpec((NG, 1), lambda i: (0, 0))],
        out_shape=[jax.ShapeDtypeStruct((NP, 1), jnp.float32),
                   jax.ShapeDtypeStruct((NG, 1), jnp.float32)],
        scratch_shapes=[pltpu.VMEM((NG, 8), jnp.float32)],
    )(Tp, bbp, oW1, ob1[None, :], oW2p, ob2p)


def kernel(z, pos, edge_index, batch, params):
    N = z.shape[0]
    E = edge_index.shape[1]
    NP = -(-N // BN) * BN
    EP = -(-E // BE) * BE

    dst = edge_index[1].astype(jnp.int32)
    posp = jnp.pad(pos, ((0, NP - N), (0, 61)))

    NZ = -(-N // (NW * CH)) * (NW * CH)
    zp = jnp.pad(z.astype(jnp.int32), (0, NZ - N))
    embp = jnp.pad(params['emb'], ((0, 0), (0, TW - H)))
    h0 = _make_gather(embp.shape[0], NZ, TW)(embp, zp)[:NP, :H]
    T = jnp.concatenate([h0, posp], axis=1)

    V = jnp.zeros((NP, 96), jnp.float32)
    eidx2 = edge_index.astype(jnp.int32).reshape(-1)
    E2 = -(-2 * E // (NW * CH)) * (NW * CH)
    eidx2 = jnp.pad(eidx2, (0, E2 - 2 * E))
    dstp = jnp.pad(dst, (0, EP - E), constant_values=NP - 1)
    zeros_np = jnp.zeros((NP, VD), jnp.float32)

    gather_T = _make_gather(NP, E2, TW)
    scatter = _make_scatter(EP, NP)

    for mpp, mixp in zip(params['mp'], params['mix']):
        rows = gather_T(T, eidx2)
        ea, eb = _edge_call(rows, E, mpp['edge'])
        parts = scatter(ea, eb, dstp, zeros_np)
        T, V = _node_call(T, V, parts, mpp, mixp)

    bbp = jnp.pad(batch.astype(jnp.int32), (0, NP - N),
                  constant_values=NG)[:, None]
    atomp, mol = _readout_call(T, bbp, params['out'])
    return (mol, atomp[:N])


# speedup vs baseline: 31.0069x; 1.1258x over previous
import functools

import jax
import jax.numpy as jnp
from jax import lax
from jax.experimental import pallas as pl
from jax.experimental.pallas import tpu as pltpu
from jax.experimental.pallas import tpu_sc as plsc

H = 64
VD = 32
NG = 64
CUTOFF = 5.0

BE = 4000
BN = 1024

NC = 2
NS = 16
NW = NC * NS
CH = 128
TW = 128


def _silu(x):
    return x * jax.nn.sigmoid(x)


def _mesh():
    return plsc.VectorSubcoreMesh(core_axis_name="c", subcore_axis_name="s",
                                  num_cores=NC, num_subcores=NS)


@functools.lru_cache(maxsize=None)
def _make_gather(TR, NI, D):
    W = NI // NW
    full, tail = divmod(W, CH)

    @functools.partial(
        pl.kernel,
        out_type=jax.ShapeDtypeStruct((NI, D), jnp.float32),
        mesh=_mesh(),
        compiler_params=pltpu.CompilerParams(use_tc_tiling_on_sc=False),
        scratch_types=[
            pltpu.VMEM((CH,), jnp.int32),
            pltpu.VMEM((CH,), jnp.int32),
            pltpu.VMEM((CH, D), jnp.float32),
            pltpu.VMEM((CH, D), jnp.float32),
            pltpu.SemaphoreType.DMA,
            pltpu.SemaphoreType.DMA,
            pltpu.SemaphoreType.DMA,
            pltpu.SemaphoreType.DMA,
            pltpu.SemaphoreType.DMA,
            pltpu.SemaphoreType.DMA,
        ],
    )
    def gath(tab, eidx, out, idx0, idx1, rows0, rows1,
             semi0, semi1, semg0, semg1, semo0, semo1):
        c = lax.axis_index("c")
        s = lax.axis_index("s")
        base = (s * NC + c) * W
        ibufs = (idx0, idx1)
        rbufs = (rows0, rows1)
        isems = (semi0, semi1)
        gsems = (semg0, semg1)
        osems = (semo0, semo1)

        def istart(g, k, n):
            pltpu.async_copy(eidx.at[pl.ds(base + g * CH, n)],
                             ibufs[k].at[pl.ds(0, n)], isems[k])

        def iwait(g, k, n):
            pltpu.make_async_copy(eidx.at[pl.ds(base + g * CH, n)],
                                  ibufs[k].at[pl.ds(0, n)], isems[k]).wait()

        def gstart(k, n):
            pltpu.async_copy(tab.at[ibufs[k].at[pl.ds(0, n)]],
                             rbufs[k].at[pl.ds(0, n), :], gsems[k])

        def gwait(k, n):
            pltpu.make_async_copy(tab.at[ibufs[k].at[pl.ds(0, n)]],
                                  rbufs[k].at[pl.ds(0, n), :],
                                  gsems[k]).wait()

        def ostart(g, k, n):
            pltpu.async_copy(rbufs[k].at[pl.ds(0, n), :],
                             out.at[pl.ds(base + g * CH, n), :], osems[k])

        def owait(g, k, n):
            pltpu.make_async_copy(rbufs[k].at[pl.ds(0, n), :],
                                  out.at[pl.ds(base + g * CH, n), :],
                                  osems[k]).wait()

        istart(0, 0, CH)

        def outer(g0, _):
            for b in range(2):
                g = g0 + b
                k = b
                k2 = (b + 1) % 2
                @pl.when(g < full)
                def _():
                    @pl.when(g >= 2)
                    def _():
                        owait(g - 2, k, CH)
                    iwait(g, k, CH)
                    gstart(k, CH)
                    @pl.when(g >= 1)
                    def _():
                        gwait(k2, CH)
                        ostart(g - 1, k2, CH)
                    @pl.when(g + 1 < full)
                    def _():
                        istart(g + 1, k2, CH)
            return _

        lax.fori_loop(0, (full + 1) // 2, lambda i, u: outer(2 * i, u), None)
        kl = (full - 1) % 2
        if full >= 2:
            owait(full - 2, (full - 2) % 2, CH)
        gwait(kl, CH)
        ostart(full - 1, kl, CH)
        owait(full - 1, kl, CH)
        if tail > 0:
            kt = full % 2
            pltpu.sync_copy(eidx.at[pl.ds(base + full * CH, tail)],
                            ibufs[kt].at[pl.ds(0, tail)])
            pltpu.async_copy(tab.at[ibufs[kt].at[pl.ds(0, tail)]],
                             rbufs[kt].at[pl.ds(0, tail), :], gsems[kt]).wait()
            pltpu.sync_copy(rbufs[kt].at[pl.ds(0, tail), :],
                            out.at[pl.ds(base + full * CH, tail), :])

    return gath


@functools.lru_cache(maxsize=None)
def _make_scatter(EP, NPAD):
    W = EP // NW
    full, tail = divmod(W, CH)
    RPT = NPAD // NS
    EPC = EP // NC

    odt = [jax.ShapeDtypeStruct((NC, NPAD, 128), jnp.float32),
           jax.ShapeDtypeStruct((NC, NPAD, VD), jnp.float32)]

    @functools.partial(
        pl.kernel,
        out_type=odt,
        mesh=_mesh(),
        compiler_params=pltpu.CompilerParams(use_tc_tiling_on_sc=False),
        scratch_types=[
            pltpu.VMEM((CH,), jnp.int32),
            pltpu.VMEM((CH,), jnp.int32),
            pltpu.VMEM((CH,), jnp.int32),
            pltpu.VMEM((CH,), jnp.int32),
            pltpu.VMEM((CH, VD), jnp.float32),
            pltpu.VMEM((CH, VD), jnp.float32),
            pltpu.VMEM((CH, VD), jnp.float32),
            pltpu.VMEM((CH, VD), jnp.float32),
            pltpu.VMEM((max(tail, 8),), jnp.int32),
            pltpu.VMEM((max(tail, 8), VD), jnp.float32),
            pltpu.VMEM_SHARED((NPAD, VD), jnp.float32),
            pltpu.SemaphoreType.DMA,
            pltpu.SemaphoreType.DMA,
            pltpu.SemaphoreType.DMA,
            pltpu.SemaphoreType.DMA,
            pltpu.SemaphoreType.DMA,
            pltpu.SemaphoreType.DMA,
            pltpu.SemaphoreType.DMA,
            pltpu.SemaphoreType.DMA,
        ],
    )
    def scat(va, vb, eidx, zeros, oa, ob,
             idx0, idx1, idx2, idx3, val0, val1, val2, val3, idxT, valT, acc,
             sl0, sl1, sl2, sl3, ss0, ss1, ss2, ss3):
        c = lax.axis_index("c")
        s = lax.axis_index("s")
        base = c * EPC + s * W
        row0 = s * RPT
        ibufs = (idx0, idx1, idx2, idx3)
        vbufs = (val0, val1, val2, val3)
        lsems = (sl0, sl1, sl2, sl3)
        ssems = (ss0, ss1, ss2, ss3)

        for p in range(5):
            v = va if p < 4 else vb
            off = (p * VD) if p < 4 else 0

            def lstart(g, k):
                b = base + g * CH
                pltpu.async_copy(eidx.at[pl.ds(b, CH)], ibufs[k], lsems[k])
                pltpu.async_copy(v.at[pl.ds(b, CH), pl.ds(off, VD)],
                                 vbufs[k], lsems[k])

            def lwait(g, k):
                b = base + g * CH
                pltpu.make_async_copy(eidx.at[pl.ds(b, CH)], ibufs[k],
                                      lsems[k]).wait()
                pltpu.make_async_copy(v.at[pl.ds(b, CH), pl.ds(off, VD)],
                                      vbufs[k], lsems[k]).wait()

            def sstart(k):
                pltpu.async_copy(vbufs[k], acc.at[ibufs[k]], ssems[k],
                                 add=True)

            def swait(k):
                pltpu.make_async_copy(vbufs[k], acc.at[ibufs[k]],
                                      ssems[k]).wait()

            pltpu.sync_copy(zeros.at[pl.ds(row0, RPT), :],
                            acc.at[pl.ds(row0, RPT), :])
            plsc.subcore_barrier()

            lstart(0, 0)
            lstart(1, 1)

            def outer(g0, _):
                for b in range(4):
                    g = g0 + b
                    k = b
                    kn = (b + 2) % 4
                    @pl.when(g < full)
                    def _():
                        lwait(g, k)
                        @pl.when(g >= 2)
                        def _():
                            swait(kn)
                        @pl.when(g + 2 < full)
                        def _():
                            lstart(g + 2, kn)
                        sstart(k)
                return _

            lax.fori_loop(0, (full + 3) // 4, lambda i, u: outer(4 * i, u),
                          None)
            if full >= 2:
                swait((full - 2) % 4)
            swait((full - 1) % 4)
            if tail > 0:
                b = base + full * CH
                pltpu.sync_copy(eidx.at[pl.ds(b, tail)], idxT)
                pltpu.sync_copy(v.at[pl.ds(b, tail), pl.ds(off, VD)], valT)
                pltpu.sync_copy(valT, acc.at[idxT], add=True)

            plsc.subcore_barrier()
            if p < 4:
                pltpu.sync_copy(acc.at[pl.ds(row0, RPT), :],
                                oa.at[c, pl.ds(row0, RPT), pl.ds(off, VD)])
            else:
                pltpu.sync_copy(acc.at[pl.ds(row0, RPT), :],
                                ob.at[c, pl.ds(row0, RPT), :])
            plsc.subcore_barrier()

    return scat


def _edge_body(hj, hi, W1a, W1b, w1c, b1, W2, b2, oa, ob):
    hjh = hj[:, :H]
    hih = hi[:, :H]
    rx = hj[:, H:H + 1] - hi[:, H:H + 1]
    ry = hj[:, H + 1:H + 2] - hi[:, H + 1:H + 2]
    rz = hj[:, H + 2:H + 3] - hi[:, H + 2:H + 3]
    d2 = rx * rx + ry * ry + rz * rz + 1e-12
    dist = jnp.sqrt(d2)
    t = hih @ W1a[...] + hjh @ W1b[...] + dist * w1c[...] + b1[...]
    t = _silu(t)
    m = t @ W2[...] + b2[...]
    x = jnp.clip(dist / CUTOFF, 0.0, 1.0)
    w = 1.0 - 6.0 * x ** 5 + 15.0 * x ** 4 - 10.0 * x ** 3
    m = m * w
    inv = 1.0 / (dist + 1e-8)
    m32 = m[:, :VD]
    oa[...] = jnp.concatenate([m, (rx * inv) * m32, (ry * inv) * m32], axis=1)
    ob[...] = jnp.concatenate(
        [(rz * inv) * m32, jnp.zeros((m.shape[0], 96), jnp.float32)], axis=1)


def _const_spec(shape):
    return pl.BlockSpec(shape, lambda i: (0,) * len(shape))


def _edge_call(rows, E, ew):
    nb = E // BE
    W1, b1 = ew[0]
    W2, b2 = ew[1]
    especs = [
        pl.BlockSpec((BE, TW), lambda i: (i, 0)),
        pl.BlockSpec((BE, TW), lambda i: (i + nb, 0)),
        _const_spec((H, H)), _const_spec((H, H)), _const_spec((1, H)),
        _const_spec((1, H)), _const_spec((H, H)), _const_spec((1, H)),
    ]
    ospec = pl.BlockSpec((BE, 128), lambda i: (i, 0))
    osh = jax.ShapeDtypeStruct((E, 128), jnp.float32)
    return pl.pallas_call(
        _edge_body,
        grid=(nb,),
        in_specs=especs,
        out_specs=[ospec, ospec],
        out_shape=[osh, osh],
    )(rows, rows, W1[:H], W1[H:2 * H], W1[2 * H:2 * H + 1], b1[None, :],
      W2, b2[None, :])


def _node_body(T, V, pa, pb,
               nW1, nb1, nW2, nb2,
               gW1, gb1, gW2, gb2,
               mW1a, mW1b, mb1, mW2, mb2,
               qW1, qb1, qW2, qb2,
               T_out, V_out):
    Tv = T[...]
    h = Tv[:, :H]
    hmv = pa[0, :, :H] + pa[1, :, :H]
    vmx = pa[0, :, H:H + VD] + pa[1, :, H:H + VD]
    vmy = pa[0, :, H + VD:] + pa[1, :, H + VD:]
    vmz = pb[0] + pb[1]
    t = _silu(hmv @ nW1[...] + nb1[...])
    hu = t @ nW2[...] + nb2[...]
    t = _silu(hmv @ gW1[...] + gb1[...])
    gate = t @ gW2[...] + gb2[...]
    sg = jax.nn.sigmoid(gate)
    Vn = V[...] + jnp.concatenate([vmx * sg, vmy * sg, vmz * sg], axis=1)
    hn = h + hu
    Vx = Vn[:, :VD]
    Vy = Vn[:, VD:2 * VD]
    Vz = Vn[:, 2 * VD:]
    vnorm = jnp.sqrt(Vx * Vx + Vy * Vy + Vz * Vz + 1e-12)
    t = _silu(hn @ mW1a[...] + vnorm @ mW1b[...] + mb1[...])
    hn = hn + (t @ mW2[...] + mb2[...])
    t = _silu(hn @ qW1[...] + qb1[...])
    g2 = t @ qW2[...] + qb2[...]
    sq = jax.nn.sigmoid(g2)
    V_out[...] = Vn * jnp.concatenate([sq, sq, sq], axis=1)
    T_out[...] = jnp.concatenate([hn, Tv[:, H:]], axis=1)


def _node_call(Tp, Vp, parts, mpp, mixp):
    NP = Tp.shape[0]
    nW1, nb1 = mpp['node'][0]
    nW2, nb2 = mpp['node'][1]
    gW1, gb1 = mpp['vgate'][0]
    gW2, gb2 = mpp['vgate'][1]
    mW1, mb1 = mixp['norm'][0]
    mW2, mb2 = mixp['norm'][1]
    qW1, qb1 = mixp['gate'][0]
    qW2, qb2 = mixp['gate'][1]
    dspec = lambda w: pl.BlockSpec((BN, w), lambda i: (i, 0))
    paspec = pl.BlockSpec((NC, BN, 128), lambda i: (0, i, 0))
    pbspec = pl.BlockSpec((NC, BN, VD), lambda i: (0, i, 0))
    specs = [dspec(TW), dspec(96), paspec, pbspec] + [
        _const_spec((H, H)), _const_spec((1, H)), _const_spec((H, H)), _const_spec((1, H)),
        _const_spec((H, H)), _const_spec((1, H)), _const_spec((H, VD)), _const_spec((1, VD)),
        _const_spec((H, H)), _const_spec((VD, H)), _const_spec((1, H)),
        _const_spec((H, H)), _const_spec((1, H)),
        _const_spec((H, H)), _const_spec((1, H)), _const_spec((H, VD)), _const_spec((1, VD))]
    return pl.pallas_call(
        _node_body,
        grid=(NP // BN,),
        in_specs=specs,
        out_specs=[dspec(TW), dspec(96)],
        out_shape=[jax.ShapeDtypeStruct((NP, TW), jnp.float32),
                   jax.ShapeDtypeStruct((NP, 96), jnp.float32)],
    )(Tp, Vp, *parts,
      nW1, nb1[None, :], nW2, nb2[None, :],
      gW1, gb1[None, :], gW2, gb2[None, :],
      mW1[:H], mW1[H:], mb1[None, :], mW2, mb2[None, :],
      qW1, qb1[None, :], qW2, qb2[None, :])


def _readout_body(T, bb, oW1, ob1, oW2, ob2, atom_out, mol_out, acc):
    i = pl.program_id(0)
    t = _silu(T[:, :H] @ oW1[...] + ob1[...])
    a8 = t @ oW2[...] + ob2[...]
    atom_out[...] = a8[:, 0:1]
    lanes = jax.lax.broadcasted_iota(jnp.int32, (BN, NG), 1)
    oh = (bb[...] == lanes).astype(jnp.float32)
    stat = jnp.concatenate(
        [a8[:, 0:1], jnp.ones((BN, 1), jnp.float32),
         jnp.zeros((BN, 6), jnp.float32)], axis=1)
    part = jax.lax.dot_general(oh, stat, (((0,), (0,)), ((), ())))

    @pl.when(i == 0)
    def _():
        acc[...] = part

    @pl.when(i != 0)
    def _():
        acc[...] = acc[...] + part

    @pl.when(i == pl.num_programs(0) - 1)
    def _():
        a = acc[...]
        mol_out[...] = a[:, 0:1] / jnp.clip(a[:, 1:2], 1.0, None)


def _readout_call(Tp, bbp, ow):
    NP = Tp.shape[0]
    oW1, ob1 = ow[0]
    oW2, ob2 = ow[1]
    oW2p = jnp.pad(oW2, ((0, 0), (0, 7)))
    ob2p = jnp.pad(ob2, (0, 7))[None, :]
    return pl.pallas_call(
        _readout_body,
        grid=(NP // BN,),
        in_specs=[pl.BlockSpec((BN, TW), lambda i: (i, 0)),
                  pl.BlockSpec((BN, 1), lambda i: (i, 0)),
                  _const_spec((H, H)), _const_spec((1, H)),
                  _const_spec((H, 8)), _const_spec((1, 8))],
        out_specs=[pl.BlockSpec((BN, 1), lambda i: (i, 0)),
                   pl.BlockSpec((NG, 1), lambda i: (0, 0))],
        out_shape=[jax.ShapeDtypeStruct((NP, 1), jnp.float32),
                   jax.ShapeDtypeStruct((NG, 1), jnp.float32)],
        scratch_shapes=[pltpu.VMEM((NG, 8), jnp.float32)],
    )(Tp, bbp, oW1, ob1[None, :], oW2p, ob2p)


def kernel(z, pos, edge_index, batch, params):
    N = z.shape[0]
    E = edge_index.shape[1]
    NP = -(-N // BN) * BN
    EP = -(-E // BE) * BE

    dst = edge_index[1].astype(jnp.int32)
    posp = jnp.pad(pos, ((0, NP - N), (0, 61)))

    NZ = -(-N // (NW * CH)) * (NW * CH)
    zp = jnp.pad(z.astype(jnp.int32), (0, NZ - N))
    embp = jnp.pad(params['emb'], ((0, 0), (0, TW - H)))
    h0 = _make_gather(embp.shape[0], NZ, TW)(embp, zp)[:NP, :H]
    T = jnp.concatenate([h0, posp], axis=1)

    V = jnp.zeros((NP, 96), jnp.float32)
    eidx2 = edge_index.astype(jnp.int32).reshape(-1)
    E2 = -(-2 * E // (NW * CH)) * (NW * CH)
    eidx2 = jnp.pad(eidx2, (0, E2 - 2 * E))
    dstp = jnp.pad(dst, (0, EP - E), constant_values=NP - 1)
    zeros_np = jnp.zeros((NP, VD), jnp.float32)

    gather_T = _make_gather(NP, E2, TW)
    scatter = _make_scatter(EP, NP)

    for mpp, mixp in zip(params['mp'], params['mix']):
        rows = gather_T(T, eidx2)
        ea, eb = _edge_call(rows, E, mpp['edge'])
        parts = scatter(ea, eb, dstp, zeros_np)
        T, V = _node_call(T, V, parts, mpp, mixp)

    bbp = jnp.pad(batch.astype(jnp.int32), (0, NP - N),
                  constant_values=NG)[:, None]
    atomp, mol = _readout_call(T, bbp, params['out'])
    return (mol, atomp[:N])
